# baseline (speedup 1.0000x reference)
import jax
import jax.numpy as jnp
from jax.experimental import pallas as pl

MAX_H = 64
MAX_W = 64


def kernel(x, pos_table):
    B, H, W, C = x.shape
    pt_r = pos_table.reshape(MAX_H, MAX_W, C)

    BB = 8

    def body(x_ref, pt_ref, o_ref):
        o_ref[...] = x_ref[...] + pt_ref[:H, :W, :][None, :, :, :]

    return pl.pallas_call(
        body,
        grid=(B // BB,),
        in_specs=[
            pl.BlockSpec((BB, H, W, C), lambda i: (i, 0, 0, 0)),
            pl.BlockSpec((MAX_H, MAX_W, C), lambda i: (0, 0, 0)),
        ],
        out_specs=pl.BlockSpec((BB, H, W, C), lambda i: (i, 0, 0, 0)),
        out_shape=jax.ShapeDtypeStruct((B, H, W, C), x.dtype),
    )(x, pt_r)


# speedup vs baseline: 1.0068x; 1.0068x over previous
import jax
import jax.numpy as jnp
from jax.experimental import pallas as pl

MAX_H = 64
MAX_W = 64


def kernel(x, pos_table):
    B, H, W, C = x.shape
    pt_r = pos_table.reshape(MAX_H, MAX_W, C)

    BB = 16

    def body(x_ref, pt_ref, o_ref):
        o_ref[...] = x_ref[...] + pt_ref[:H, :W, :][None, :, :, :]

    return pl.pallas_call(
        body,
        grid=(B // BB,),
        in_specs=[
            pl.BlockSpec((BB, H, W, C), lambda i: (i, 0, 0, 0)),
            pl.BlockSpec((MAX_H, MAX_W, C), lambda i: (0, 0, 0)),
        ],
        out_specs=pl.BlockSpec((BB, H, W, C), lambda i: (i, 0, 0, 0)),
        out_shape=jax.ShapeDtypeStruct((B, H, W, C), x.dtype),
    )(x, pt_r)
